# Initial kernel scaffold; baseline (speedup 1.0000x reference)
import functools

import jax
import jax.numpy as jnp
from jax import lax
from jax.experimental import pallas as pl
from jax.experimental.pallas import tpu as pltpu
from jax.experimental.pallas import tpu_sc as plsc

B, S, D, F, E, K = 4, 2048, 1024, 4096, 8, 2
N = B * S
R = N * K
T = 512
NT = (R + E * (T - 1) + T - 1) // T
NP = NT * T
F_BLK = 1024
NJ = F // F_BLK
NW = 32
RB = 1024


def _router_body(x_ref, wg_ref, ei_ref, w_ref):
    l = lax.dot_general(x_ref[...], wg_ref[...], (((1,), (1,)), ((), ())),
                        preferred_element_type=jnp.float32,
                        precision=lax.Precision.HIGHEST)
    ie = lax.broadcasted_iota(jnp.int32, (RB, E), 1)
    m1 = jnp.max(l, axis=1, keepdims=True)
    i1 = jnp.min(jnp.where(l == m1, ie, E), axis=1, keepdims=True)
    l2 = jnp.where(ie == i1, -jnp.inf, l)
    m2 = jnp.max(l2, axis=1, keepdims=True)
    i2 = jnp.min(jnp.where(l2 == m2, ie, E), axis=1, keepdims=True)
    t = jnp.exp(m2 - m1)
    s = 1.0 + t
    ei_ref[...] = jnp.concatenate([i1, i2], axis=1)
    w_ref[...] = jnp.concatenate([1.0 / s, t / s], axis=1)


def _router(xr, Wg):
    return pl.pallas_call(
        _router_body,
        grid=(N // RB,),
        in_specs=[
            pl.BlockSpec((RB, D), lambda i: (i, 0)),
            pl.BlockSpec((E, D), lambda i: (0, 0)),
        ],
        out_specs=[
            pl.BlockSpec((RB, K), lambda i: (i, 0)),
            pl.BlockSpec((RB, K), lambda i: (i, 0)),
        ],
        out_shape=[
            jax.ShapeDtypeStruct((N, K), jnp.int32),
            jax.ShapeDtypeStruct((N, K), jnp.float32),
        ],
    )(xr, Wg)


_RW = R // NW
_CH = 64
_NCH = _RW // _CH


def _dispatch_sc(xb, tok, pos, wf):
    mesh = plsc.VectorSubcoreMesh(core_axis_name="c", subcore_axis_name="s")

    @functools.partial(
        pl.kernel,
        out_type=[
            jax.ShapeDtypeStruct((NP, E, 128), jnp.bfloat16),
            jax.ShapeDtypeStruct((NP, 1), jnp.float32),
        ],
        mesh=mesh,
        scratch_types=[
            pltpu.VMEM((_CH,), jnp.int32),
            pltpu.VMEM((_CH,), jnp.int32),
            pltpu.VMEM((_CH, 1), jnp.float32),
            pltpu.VMEM((_CH, E, 128), jnp.bfloat16),
            pltpu.SemaphoreType.DMA,
        ],
    )
    def k(xb_hbm, tok_hbm, pos_hbm, wf_hbm, xs_hbm, rw_hbm,
          tokv, posv, wv, rows, sem):
        wid = lax.axis_index("s") * 2 + lax.axis_index("c")
        base = wid * _RW

        def body(c, carry):
            off = base + c * _CH
            pltpu.sync_copy(tok_hbm.at[pl.ds(off, _CH)], tokv)
            pltpu.sync_copy(pos_hbm.at[pl.ds(off, _CH)], posv)
            pltpu.sync_copy(wf_hbm.at[pl.ds(off, _CH)], wv)
            pltpu.async_copy(xb_hbm.at[tokv], rows, sem).wait()
            pltpu.async_copy(rows, xs_hbm.at[posv], sem).wait()
            pltpu.async_copy(wv, rw_hbm.at[posv], sem).wait()
            return carry

        lax.fori_loop(0, _NCH, body, 0)

    return k(xb, tok, pos, wf)


def _gmm_body(te_ref, na_ref, xs_ref, w1_ref, w3_ref, w2_ref, b3_ref, rw_ref,
              out_ref, acc_ref):
    i = pl.program_id(0)
    j = pl.program_id(1)

    @pl.when(i < na_ref[0])
    def _():
        xb = xs_ref[...]
        u = lax.dot_general(xb, w1_ref[0], (((1,), (1,)), ((), ())),
                            preferred_element_type=jnp.float32)
        v = lax.dot_general(xb, w3_ref[0], (((1,), (1,)), ((), ())),
                            preferred_element_type=jnp.float32)
        h = (u * jax.nn.sigmoid(u)) * (v + b3_ref[0])
        p = lax.dot_general(h.astype(jnp.bfloat16), w2_ref[0],
                            (((1,), (1,)), ((), ())),
                            preferred_element_type=jnp.float32)

        @pl.when(j == 0)
        def _():
            acc_ref[...] = p

        @pl.when(j > 0)
        def _():
            acc_ref[...] += p

        @pl.when(j == NJ - 1)
        def _():
            out_ref[...] = acc_ref[...] * rw_ref[...]


def _gmm(xs, W1b, W3b, W2b, b3r, rw, te, na):
    def ic(i, na_ref):
        return jnp.minimum(i, na_ref[0] - 1)

    grid_spec = pltpu.PrefetchScalarGridSpec(
        num_scalar_prefetch=2,
        grid=(NT, NJ),
        in_specs=[
            pl.BlockSpec((T, D), lambda i, j, te, na: (ic(i, na), 0)),
            pl.BlockSpec((1, F_BLK, D), lambda i, j, te, na: (te[ic(i, na)], j, 0)),
            pl.BlockSpec((1, F_BLK, D), lambda i, j, te, na: (te[ic(i, na)], j, 0)),
            pl.BlockSpec((1, D, F_BLK), lambda i, j, te, na: (te[ic(i, na)], 0, j)),
            pl.BlockSpec((1, 1, F_BLK), lambda i, j, te, na: (te[ic(i, na)], 0, j)),
            pl.BlockSpec((T, 1), lambda i, j, te, na: (ic(i, na), 0)),
        ],
        out_specs=pl.BlockSpec((T, D), lambda i, j, te, na: (ic(i, na), 0)),
        scratch_shapes=[pltpu.VMEM((T, D), jnp.float32)],
    )
    return pl.pallas_call(
        _gmm_body,
        grid_spec=grid_spec,
        out_shape=jax.ShapeDtypeStruct((NP, D), jnp.float32),
        compiler_params=pltpu.CompilerParams(
            dimension_semantics=("arbitrary", "arbitrary")),
    )(te, na, xs, W1b, W3b, W2b, b3r, rw)


_TN = N // NW
_CH2 = 32
_NCH2 = _TN // _CH2


def _combine_sc(rows_mat, pos0, pos1):
    mesh = plsc.VectorSubcoreMesh(core_axis_name="c", subcore_axis_name="s")

    @functools.partial(
        pl.kernel,
        out_type=jax.ShapeDtypeStruct((N, D), jnp.float32),
        mesh=mesh,
        scratch_types=[
            pltpu.VMEM((_CH2,), jnp.int32),
            pltpu.VMEM((_CH2,), jnp.int32),
            pltpu.VMEM((_CH2, D), jnp.float32),
            pltpu.VMEM((_CH2, D), jnp.float32),
            pltpu.SemaphoreType.DMA,
        ],
    )
    def k(rows_hbm, p0_hbm, p1_hbm, out_hbm, i0v, i1v, r0, r1, sem):
        wid = lax.axis_index("s") * 2 + lax.axis_index("c")
        base = wid * _TN

        def chunk(c, carry):
            off = base + c * _CH2
            pltpu.sync_copy(p0_hbm.at[pl.ds(off, _CH2)], i0v)
            pltpu.sync_copy(p1_hbm.at[pl.ds(off, _CH2)], i1v)
            pltpu.async_copy(rows_hbm.at[i0v], r0, sem).wait()
            pltpu.async_copy(rows_hbm.at[i1v], r1, sem).wait()

            def tokrow(j, carry2):
                for cc in range(D // 16):
                    sl = pl.ds(cc * 16, 16)
                    r0[j, sl] = r0[j, sl] + r1[j, sl]
                return carry2

            lax.fori_loop(0, _CH2, tokrow, 0)
            pltpu.sync_copy(r0, out_hbm.at[pl.ds(off, _CH2)])
            return carry

        lax.fori_loop(0, _NCH2, chunk, 0)

    return k(rows_mat, pos0, pos1)


def kernel(x, W1, W2, W3, b3, Wg):
    xr = x.reshape(N, D)
    xb = xr.astype(jnp.bfloat16).reshape(N, E, 128)
    W1b = W1.astype(jnp.bfloat16)
    W3b = W3.astype(jnp.bfloat16)
    W2b = W2.astype(jnp.bfloat16)
    b3r = b3.reshape(E, 1, F)

    eidx, wts = _router(xr, Wg)

    ef = eidx.reshape(R)
    wf = wts.reshape(R, 1)
    onehot = (ef[:, None] == jnp.arange(E, dtype=jnp.int32)[None, :]).astype(jnp.int32)
    csum = jnp.cumsum(onehot, axis=0)
    counts = csum[-1]
    rank = jnp.sum(onehot * (csum - 1), axis=1)
    padded = ((counts + T - 1) // T) * T
    pend = jnp.cumsum(padded)
    poff = pend - padded
    pos = jnp.sum(onehot * poff[None, :], axis=1) + rank
    tile_i = jnp.arange(NT, dtype=jnp.int32)
    te = jnp.minimum(
        jnp.sum((tile_i[:, None] * T >= pend[None, :]).astype(jnp.int32), axis=1),
        E - 1).astype(jnp.int32)
    na = (pend[-1] // T).astype(jnp.int32).reshape(1)
    tok = jnp.arange(R, dtype=jnp.int32) // K
    pos2 = pos.reshape(N, K)
    pos0 = pos2[:, 0]
    pos1 = pos2[:, 1]

    xs, rw = _dispatch_sc(xb, tok, pos, wf)

    rows_mat = _gmm(xs.reshape(NP, D), W1b, W3b, W2b, b3r, rw, te, na)

    out = _combine_sc(rows_mat, pos0, pos1)
    return out.reshape(B, S, D)


# speedup vs baseline: 1.5224x; 1.5224x over previous
import functools

import jax
import jax.numpy as jnp
from jax import lax
from jax.experimental import pallas as pl
from jax.experimental.pallas import tpu as pltpu
from jax.experimental.pallas import tpu_sc as plsc

B, S, D, F, E, K = 4, 2048, 1024, 4096, 8, 2
N = B * S
R = N * K
T = 512
NT = (R + E * (T - 1) + T - 1) // T
NP = NT * T
F_BLK = 1024
NJ = F // F_BLK
NW = 32
RB = 1024


def _router_body(x_ref, wg_ref, ei_ref, w_ref):
    l = lax.dot_general(x_ref[...].astype(jnp.bfloat16),
                        wg_ref[...].astype(jnp.bfloat16),
                        (((1,), (1,)), ((), ())),
                        preferred_element_type=jnp.float32)
    ie = lax.broadcasted_iota(jnp.int32, (RB, E), 1)
    m1 = jnp.max(l, axis=1, keepdims=True)
    i1 = jnp.min(jnp.where(l == m1, ie, E), axis=1, keepdims=True)
    l2 = jnp.where(ie == i1, -jnp.inf, l)
    m2 = jnp.max(l2, axis=1, keepdims=True)
    i2 = jnp.min(jnp.where(l2 == m2, ie, E), axis=1, keepdims=True)
    t = jnp.exp(m2 - m1)
    s = 1.0 + t
    ei_ref[...] = jnp.concatenate([i1, i2], axis=1)
    w_ref[...] = jnp.concatenate([1.0 / s, t / s], axis=1)


def _router(xr, Wg):
    return pl.pallas_call(
        _router_body,
        grid=(N // RB,),
        in_specs=[
            pl.BlockSpec((RB, D), lambda i: (i, 0)),
            pl.BlockSpec((E, D), lambda i: (0, 0)),
        ],
        out_specs=[
            pl.BlockSpec((RB, K), lambda i: (i, 0)),
            pl.BlockSpec((RB, K), lambda i: (i, 0)),
        ],
        out_shape=[
            jax.ShapeDtypeStruct((N, K), jnp.int32),
            jax.ShapeDtypeStruct((N, K), jnp.float32),
        ],
    )(xr, Wg)


_RW = R // NW
_CH = 64
_NCH = _RW // _CH


def _dispatch_sc(xb, tok, pos):
    mesh = plsc.VectorSubcoreMesh(core_axis_name="c", subcore_axis_name="s")

    @functools.partial(
        pl.kernel,
        out_type=jax.ShapeDtypeStruct((NP, D // 2), jnp.int32),
        mesh=mesh,
        scratch_types=[
            pltpu.VMEM((_CH,), jnp.int32),
            pltpu.VMEM((_CH,), jnp.int32),
            pltpu.VMEM((_CH, D // 2), jnp.int32),
            pltpu.SemaphoreType.DMA,
        ],
    )
    def k(xb_hbm, tok_hbm, pos_hbm, xs_hbm, tokv, posv, rows, sem):
        wid = lax.axis_index("s") * 2 + lax.axis_index("c")
        base = wid * _RW

        def body(c, carry):
            off = base + c * _CH
            pltpu.sync_copy(tok_hbm.at[pl.ds(off, _CH)], tokv)
            pltpu.sync_copy(pos_hbm.at[pl.ds(off, _CH)], posv)
            pltpu.async_copy(xb_hbm.at[tokv], rows, sem).wait()
            pltpu.async_copy(rows, xs_hbm.at[posv], sem).wait()
            return carry

        lax.fori_loop(0, _NCH, body, 0)

    return k(xb, tok, pos)


def _gmm_body(te_ref, na_ref, xs_ref, w1_ref, w3_ref, w2_ref, b3_ref,
              out_ref, acc_ref):
    i = pl.program_id(0)
    j = pl.program_id(1)

    @pl.when(i < na_ref[0])
    def _():
        xb = xs_ref[...]
        u = lax.dot_general(xb, w1_ref[0], (((1,), (1,)), ((), ())),
                            preferred_element_type=jnp.float32)
        v = lax.dot_general(xb, w3_ref[0], (((1,), (1,)), ((), ())),
                            preferred_element_type=jnp.float32)
        h = (u * jax.nn.sigmoid(u)) * (v + b3_ref[0])
        p = lax.dot_general(h.astype(jnp.bfloat16), w2_ref[0],
                            (((1,), (1,)), ((), ())),
                            preferred_element_type=jnp.float32)

        @pl.when(j == 0)
        def _():
            acc_ref[...] = p

        @pl.when(j > 0)
        def _():
            acc_ref[...] += p

        @pl.when(j == NJ - 1)
        def _():
            out_ref[...] = acc_ref[...]


def _gmm(xs, W1b, W3b, W2b, b3r, te, na):
    def ic(i, na_ref):
        return jnp.minimum(i, na_ref[0] - 1)

    grid_spec = pltpu.PrefetchScalarGridSpec(
        num_scalar_prefetch=2,
        grid=(NT, NJ),
        in_specs=[
            pl.BlockSpec((T, D), lambda i, j, te, na: (ic(i, na), 0)),
            pl.BlockSpec((1, F_BLK, D), lambda i, j, te, na: (te[ic(i, na)], j, 0)),
            pl.BlockSpec((1, F_BLK, D), lambda i, j, te, na: (te[ic(i, na)], j, 0)),
            pl.BlockSpec((1, D, F_BLK), lambda i, j, te, na: (te[ic(i, na)], 0, j)),
            pl.BlockSpec((1, 1, F_BLK), lambda i, j, te, na: (te[ic(i, na)], 0, j)),
        ],
        out_specs=pl.BlockSpec((T, D), lambda i, j, te, na: (ic(i, na), 0)),
        scratch_shapes=[pltpu.VMEM((T, D), jnp.float32)],
    )
    return pl.pallas_call(
        _gmm_body,
        grid_spec=grid_spec,
        out_shape=jax.ShapeDtypeStruct((NP, D), jnp.float32),
        compiler_params=pltpu.CompilerParams(
            dimension_semantics=("arbitrary", "arbitrary")),
    )(te, na, xs, W1b, W3b, W2b, b3r)


_TN = N // NW
_CH2 = 32
_NCH2 = _TN // _CH2


def _combine_sc(rows_mat, pos0, pos1, w0, w1):
    mesh = plsc.VectorSubcoreMesh(core_axis_name="c", subcore_axis_name="s")

    @functools.partial(
        pl.kernel,
        out_type=jax.ShapeDtypeStruct((N, D), jnp.float32),
        mesh=mesh,
        scratch_types=[
            pltpu.VMEM((_CH2,), jnp.int32),
            pltpu.VMEM((_CH2,), jnp.int32),
            pltpu.VMEM((_CH2, 16), jnp.float32),
            pltpu.VMEM((_CH2, 16), jnp.float32),
            pltpu.VMEM((_CH2, D), jnp.float32),
            pltpu.VMEM((_CH2, D), jnp.float32),
            pltpu.SemaphoreType.DMA,
        ],
    )
    def k(rows_hbm, p0_hbm, p1_hbm, w0_hbm, w1_hbm, out_hbm,
          i0v, i1v, w0v, w1v, r0, r1, sem):
        wid = lax.axis_index("s") * 2 + lax.axis_index("c")
        base = wid * _TN

        def chunk(c, carry):
            off = base + c * _CH2
            pltpu.sync_copy(p0_hbm.at[pl.ds(off, _CH2)], i0v)
            pltpu.sync_copy(p1_hbm.at[pl.ds(off, _CH2)], i1v)
            pltpu.sync_copy(w0_hbm.at[pl.ds(off, _CH2)], w0v)
            pltpu.sync_copy(w1_hbm.at[pl.ds(off, _CH2)], w1v)
            pltpu.async_copy(rows_hbm.at[i0v], r0, sem).wait()
            pltpu.async_copy(rows_hbm.at[i1v], r1, sem).wait()

            def tokrow(j, carry2):
                w0s = w0v[j, pl.ds(0, 16)]
                w1s = w1v[j, pl.ds(0, 16)]
                for cc in range(D // 16):
                    sl = pl.ds(cc * 16, 16)
                    r0[j, sl] = r0[j, sl] * w0s + r1[j, sl] * w1s
                return carry2

            lax.fori_loop(0, _CH2, tokrow, 0)
            pltpu.sync_copy(r0, out_hbm.at[pl.ds(off, _CH2)])
            return carry

        lax.fori_loop(0, _NCH2, chunk, 0)

    return k(rows_mat, pos0, pos1, w0, w1)


def kernel(x, W1, W2, W3, b3, Wg):
    xr = x.reshape(N, D)
    xb = lax.bitcast_convert_type(
        xr.astype(jnp.bfloat16).reshape(N, D // 2, 2), jnp.int32)
    W1b = W1.astype(jnp.bfloat16)
    W3b = W3.astype(jnp.bfloat16)
    W2b = W2.astype(jnp.bfloat16)
    b3r = b3.reshape(E, 1, F)

    eidx, wts = _router(xr, Wg)

    ef = eidx.reshape(R)
    wf = wts.reshape(R, 1)
    onehot = (ef[:, None] == jnp.arange(E, dtype=jnp.int32)[None, :]).astype(jnp.int32)
    csum = jnp.cumsum(onehot, axis=0)
    counts = csum[-1]
    rank = jnp.sum(onehot * (csum - 1), axis=1)
    padded = ((counts + T - 1) // T) * T
    pend = jnp.cumsum(padded)
    poff = pend - padded
    pos = jnp.sum(onehot * poff[None, :], axis=1) + rank
    tile_i = jnp.arange(NT, dtype=jnp.int32)
    te = jnp.minimum(
        jnp.sum((tile_i[:, None] * T >= pend[None, :]).astype(jnp.int32), axis=1),
        E - 1).astype(jnp.int32)
    na = (pend[-1] // T).astype(jnp.int32).reshape(1)
    tok = jnp.arange(R, dtype=jnp.int32) // K
    pos2 = pos.reshape(N, K)
    pos0 = pos2[:, 0]
    pos1 = pos2[:, 1]
    w2d = wf.reshape(N, K)
    w0 = jnp.broadcast_to(w2d[:, 0:1], (N, 16))
    w1 = jnp.broadcast_to(w2d[:, 1:2], (N, 16))

    xs = _dispatch_sc(xb, tok, pos)
    xs_bf = lax.bitcast_convert_type(xs, jnp.bfloat16).reshape(NP, D)

    rows_mat = _gmm(xs_bf, W1b, W3b, W2b, b3r, te, na)

    out = _combine_sc(rows_mat, pos0, pos1, w0, w1)
    return out.reshape(B, S, D)


# speedup vs baseline: 2.3925x; 1.5715x over previous
import functools

import jax
import jax.numpy as jnp
from jax import lax
from jax.experimental import pallas as pl
from jax.experimental.pallas import tpu as pltpu
from jax.experimental.pallas import tpu_sc as plsc

B, S, D, F, E, K = 4, 2048, 1024, 4096, 8, 2
N = B * S
R = N * K
T = 512
NT = (R + E * (T - 1) + T - 1) // T
NP = NT * T
F_BLK = 1024
NJ = F // F_BLK
NW = 32
RB = 1024


def _router_body(x_ref, wg_ref, ei_ref, w0_ref, w1_ref):
    l = lax.dot_general(x_ref[...].astype(jnp.bfloat16),
                        wg_ref[...].astype(jnp.bfloat16),
                        (((1,), (1,)), ((), ())),
                        preferred_element_type=jnp.float32)
    ie = lax.broadcasted_iota(jnp.int32, (RB, E), 1)
    m1 = jnp.max(l, axis=1, keepdims=True)
    i1 = jnp.min(jnp.where(l == m1, ie, E), axis=1, keepdims=True)
    l2 = jnp.where(ie == i1, -jnp.inf, l)
    m2 = jnp.max(l2, axis=1, keepdims=True)
    i2 = jnp.min(jnp.where(l2 == m2, ie, E), axis=1, keepdims=True)
    t = jnp.exp(m2 - m1)
    s = 1.0 + t
    ei_ref[...] = jnp.concatenate([i1, i2], axis=1)
    w0_ref[...] = jnp.broadcast_to(1.0 / s, (RB, 16))
    w1_ref[...] = jnp.broadcast_to(t / s, (RB, 16))


def _router(xr, Wg):
    return pl.pallas_call(
        _router_body,
        grid=(N // RB,),
        in_specs=[
            pl.BlockSpec((RB, D), lambda i: (i, 0)),
            pl.BlockSpec((E, D), lambda i: (0, 0)),
        ],
        out_specs=[
            pl.BlockSpec((RB, K), lambda i: (i, 0)),
            pl.BlockSpec((RB, 16), lambda i: (i, 0)),
            pl.BlockSpec((RB, 16), lambda i: (i, 0)),
        ],
        out_shape=[
            jax.ShapeDtypeStruct((N, K), jnp.int32),
            jax.ShapeDtypeStruct((N, 16), jnp.float32),
            jax.ShapeDtypeStruct((N, 16), jnp.float32),
        ],
    )(xr, Wg)


_RW = R // NW
_CH = 64
_NCH = _RW // _CH


def _dispatch_sc(xb, tok, pos):
    mesh = plsc.VectorSubcoreMesh(core_axis_name="c", subcore_axis_name="s")

    @functools.partial(
        pl.kernel,
        out_type=jax.ShapeDtypeStruct((NP, D), jnp.float32),
        mesh=mesh,
        scratch_types=[
            pltpu.VMEM((_CH,), jnp.int32),
            pltpu.VMEM((_CH,), jnp.int32),
            pltpu.VMEM((_CH, D), jnp.float32),
            pltpu.SemaphoreType.DMA,
        ],
    )
    def k(xb_hbm, tok_hbm, pos_hbm, xs_hbm, tokv, posv, rows, sem):
        wid = lax.axis_index("s") * 2 + lax.axis_index("c")
        base = wid * _RW

        def body(c, carry):
            off = base + c * _CH
            pltpu.sync_copy(tok_hbm.at[pl.ds(off, _CH)], tokv)
            pltpu.sync_copy(pos_hbm.at[pl.ds(off, _CH)], posv)
            pltpu.async_copy(xb_hbm.at[tokv], rows, sem).wait()
            pltpu.async_copy(rows, xs_hbm.at[posv], sem).wait()
            return carry

        lax.fori_loop(0, _NCH, body, 0)

    return k(xb, tok, pos)


def _gmm_body(te_ref, na_ref, xs_ref, w1_ref, w3_ref, w2_ref, b3_ref,
              out_ref, acc_ref, xb_ref):
    i = pl.program_id(0)
    j = pl.program_id(1)

    @pl.when((i < na_ref[0]) & (j == 0))
    def _():
        xb_ref[...] = xs_ref[...].astype(jnp.bfloat16)

    @pl.when(i < na_ref[0])
    def _():
        xb = xb_ref[...]
        u = lax.dot_general(xb, w1_ref[0], (((1,), (1,)), ((), ())),
                            preferred_element_type=jnp.float32)
        v = lax.dot_general(xb, w3_ref[0], (((1,), (1,)), ((), ())),
                            preferred_element_type=jnp.float32)
        h = (u * jax.nn.sigmoid(u)) * (v + b3_ref[0])
        p = lax.dot_general(h.astype(jnp.bfloat16), w2_ref[0],
                            (((1,), (1,)), ((), ())),
                            preferred_element_type=jnp.float32)

        @pl.when(j == 0)
        def _():
            acc_ref[...] = p

        @pl.when(j > 0)
        def _():
            acc_ref[...] += p

        @pl.when(j == NJ - 1)
        def _():
            out_ref[...] = acc_ref[...]


def _gmm(xs, W1b, W3b, W2b, b3r, te, na):
    def ic(i, na_ref):
        return jnp.minimum(i, na_ref[0] - 1)

    grid_spec = pltpu.PrefetchScalarGridSpec(
        num_scalar_prefetch=2,
        grid=(NT, NJ),
        in_specs=[
            pl.BlockSpec((T, D), lambda i, j, te, na: (ic(i, na), 0)),
            pl.BlockSpec((1, F_BLK, D), lambda i, j, te, na: (te[ic(i, na)], j, 0)),
            pl.BlockSpec((1, F_BLK, D), lambda i, j, te, na: (te[ic(i, na)], j, 0)),
            pl.BlockSpec((1, D, F_BLK), lambda i, j, te, na: (te[ic(i, na)], 0, j)),
            pl.BlockSpec((1, 1, F_BLK), lambda i, j, te, na: (te[ic(i, na)], 0, j)),
        ],
        out_specs=pl.BlockSpec((T, D), lambda i, j, te, na: (ic(i, na), 0)),
        scratch_shapes=[pltpu.VMEM((T, D), jnp.float32),
                        pltpu.VMEM((T, D), jnp.bfloat16)],
    )
    return pl.pallas_call(
        _gmm_body,
        grid_spec=grid_spec,
        out_shape=jax.ShapeDtypeStruct((NP, D), jnp.float32),
        compiler_params=pltpu.CompilerParams(
            dimension_semantics=("arbitrary", "arbitrary")),
    )(te, na, xs, W1b, W3b, W2b, b3r)


_TN = N // NW
_CH2 = 32
_NCH2 = _TN // _CH2


def _combine_sc(rows_mat, pos0, pos1, w0, w1):
    mesh = plsc.VectorSubcoreMesh(core_axis_name="c", subcore_axis_name="s")

    @functools.partial(
        pl.kernel,
        out_type=jax.ShapeDtypeStruct((N, D), jnp.float32),
        mesh=mesh,
        scratch_types=[
            pltpu.VMEM((_CH2,), jnp.int32),
            pltpu.VMEM((_CH2,), jnp.int32),
            pltpu.VMEM((_CH2, 16), jnp.float32),
            pltpu.VMEM((_CH2, 16), jnp.float32),
            pltpu.VMEM((_CH2, D), jnp.float32),
            pltpu.VMEM((_CH2, D), jnp.float32),
            pltpu.SemaphoreType.DMA,
        ],
    )
    def k(rows_hbm, p0_hbm, p1_hbm, w0_hbm, w1_hbm, out_hbm,
          i0v, i1v, w0v, w1v, r0, r1, sem):
        wid = lax.axis_index("s") * 2 + lax.axis_index("c")
        base = wid * _TN

        def chunk(c, carry):
            off = base + c * _CH2
            pltpu.sync_copy(p0_hbm.at[pl.ds(off, _CH2)], i0v)
            pltpu.sync_copy(p1_hbm.at[pl.ds(off, _CH2)], i1v)
            pltpu.sync_copy(w0_hbm.at[pl.ds(off, _CH2)], w0v)
            pltpu.sync_copy(w1_hbm.at[pl.ds(off, _CH2)], w1v)
            pltpu.async_copy(rows_hbm.at[i0v], r0, sem).wait()
            pltpu.async_copy(rows_hbm.at[i1v], r1, sem).wait()

            def tokrow(j, carry2):
                w0s = w0v[j, pl.ds(0, 16)]
                w1s = w1v[j, pl.ds(0, 16)]
                for cc in range(D // 16):
                    sl = pl.ds(cc * 16, 16)
                    r0[j, sl] = r0[j, sl] * w0s + r1[j, sl] * w1s
                return carry2

            lax.fori_loop(0, _CH2, tokrow, 0)
            pltpu.sync_copy(r0, out_hbm.at[pl.ds(off, _CH2)])
            return carry

        lax.fori_loop(0, _NCH2, chunk, 0)

    return k(rows_mat, pos0, pos1, w0, w1)


def kernel(x, W1, W2, W3, b3, Wg):
    xr = x.reshape(N, D)
    W1b = W1.astype(jnp.bfloat16)
    W3b = W3.astype(jnp.bfloat16)
    W2b = W2.astype(jnp.bfloat16)
    b3r = b3.reshape(E, 1, F)

    eidx, w0, w1 = _router(xr, Wg)

    ef = eidx.reshape(1, R)
    onehot = (ef == jnp.arange(E, dtype=jnp.int32)[:, None]).astype(jnp.int32)
    csum = jnp.cumsum(onehot, axis=1)
    counts = csum[:, -1]
    rank = jnp.sum(onehot * (csum - 1), axis=0)
    padded = ((counts + T - 1) // T) * T
    pend = jnp.cumsum(padded)
    poff = pend - padded
    pos = jnp.sum(onehot * poff[:, None], axis=0) + rank
    tile_i = jnp.arange(NT, dtype=jnp.int32)
    te = jnp.minimum(
        jnp.sum((tile_i[None, :] * T >= pend[:, None]).astype(jnp.int32), axis=0),
        E - 1).astype(jnp.int32)
    na = (pend[-1] // T).astype(jnp.int32).reshape(1)
    tok = jnp.arange(R, dtype=jnp.int32) // K
    pos2 = pos.reshape(N, K)
    pos0 = pos2[:, 0]
    pos1 = pos2[:, 1]

    xs = _dispatch_sc(xr, tok, pos)

    rows_mat = _gmm(xs, W1b, W3b, W2b, b3r, te, na)

    out = _combine_sc(rows_mat, pos0, pos1, w0, w1)
    return out.reshape(B, S, D)


# speedup vs baseline: 2.5881x; 1.0818x over previous
import functools

import jax
import jax.numpy as jnp
from jax import lax
from jax.experimental import pallas as pl
from jax.experimental.pallas import tpu as pltpu
from jax.experimental.pallas import tpu_sc as plsc

B, S, D, F, E, K = 4, 2048, 1024, 4096, 8, 2
N = B * S
R = N * K
T = 512
NT = (R + E * (T - 1) + T - 1) // T
NP = NT * T
F_BLK = 1024
NJ = F // F_BLK
NW = 32
RB = 1024


def _router_body(x_ref, wg_ref, ei_ref, w0_ref, w1_ref):
    l = lax.dot_general(x_ref[...].astype(jnp.bfloat16),
                        wg_ref[...].astype(jnp.bfloat16),
                        (((1,), (1,)), ((), ())),
                        preferred_element_type=jnp.float32)
    ie = lax.broadcasted_iota(jnp.int32, (RB, E), 1)
    m1 = jnp.max(l, axis=1, keepdims=True)
    i1 = jnp.min(jnp.where(l == m1, ie, E), axis=1, keepdims=True)
    l2 = jnp.where(ie == i1, -jnp.inf, l)
    m2 = jnp.max(l2, axis=1, keepdims=True)
    i2 = jnp.min(jnp.where(l2 == m2, ie, E), axis=1, keepdims=True)
    t = jnp.exp(m2 - m1)
    s = 1.0 + t
    ei_ref[...] = jnp.concatenate([i1, i2], axis=1)
    w0_ref[...] = jnp.broadcast_to(1.0 / s, (RB, 16))
    w1_ref[...] = jnp.broadcast_to(t / s, (RB, 16))


def _router(xr, Wg):
    return pl.pallas_call(
        _router_body,
        grid=(N // RB,),
        in_specs=[
            pl.BlockSpec((RB, D), lambda i: (i, 0)),
            pl.BlockSpec((E, D), lambda i: (0, 0)),
        ],
        out_specs=[
            pl.BlockSpec((RB, K), lambda i: (i, 0)),
            pl.BlockSpec((RB, 16), lambda i: (i, 0)),
            pl.BlockSpec((RB, 16), lambda i: (i, 0)),
        ],
        out_shape=[
            jax.ShapeDtypeStruct((N, K), jnp.int32),
            jax.ShapeDtypeStruct((N, 16), jnp.float32),
            jax.ShapeDtypeStruct((N, 16), jnp.float32),
        ],
    )(xr, Wg)


_RW = R // NW
_CH = 64
_NCH = _RW // _CH


def _dispatch_sc(xb, tok, pos):
    mesh = plsc.VectorSubcoreMesh(core_axis_name="c", subcore_axis_name="s")

    @functools.partial(
        pl.kernel,
        out_type=jax.ShapeDtypeStruct((NP, D), jnp.float32),
        mesh=mesh,
        scratch_types=[
            pltpu.VMEM((_CH,), jnp.int32),
            pltpu.VMEM((_CH,), jnp.int32),
            pltpu.VMEM((_CH, D), jnp.float32),
            pltpu.SemaphoreType.DMA,
        ],
    )
    def k(xb_hbm, tok_hbm, pos_hbm, xs_hbm, tokv, posv, rows, sem):
        wid = lax.axis_index("s") * 2 + lax.axis_index("c")
        base = wid * _RW

        def body(c, carry):
            off = base + c * _CH
            pltpu.sync_copy(tok_hbm.at[pl.ds(off, _CH)], tokv)
            pltpu.sync_copy(pos_hbm.at[pl.ds(off, _CH)], posv)
            pltpu.async_copy(xb_hbm.at[tokv], rows, sem).wait()
            pltpu.async_copy(rows, xs_hbm.at[posv], sem).wait()
            return carry

        lax.fori_loop(0, _NCH, body, 0)

    return k(xb, tok, pos)


def _gmm_body(te_ref, na_ref, xs_ref, w1_ref, w3_ref, w2_ref, b3_ref,
              out_ref, acc_ref, xb_ref):
    i = pl.program_id(0)
    j = pl.program_id(1)

    @pl.when((i < na_ref[0]) & (j == 0))
    def _():
        xb_ref[...] = xs_ref[...].astype(jnp.bfloat16)

    @pl.when(i < na_ref[0])
    def _():
        xb = xb_ref[...]
        u = lax.dot_general(xb, w1_ref[0].astype(jnp.bfloat16),
                            (((1,), (1,)), ((), ())),
                            preferred_element_type=jnp.float32)
        v = lax.dot_general(xb, w3_ref[0].astype(jnp.bfloat16),
                            (((1,), (1,)), ((), ())),
                            preferred_element_type=jnp.float32)
        h = (u * jax.nn.sigmoid(u)) * (v + b3_ref[0])
        p = lax.dot_general(h.astype(jnp.bfloat16), w2_ref[0].astype(jnp.bfloat16),
                            (((1,), (1,)), ((), ())),
                            preferred_element_type=jnp.float32)

        @pl.when(j == 0)
        def _():
            acc_ref[...] = p

        @pl.when(j > 0)
        def _():
            acc_ref[...] += p

        @pl.when(j == NJ - 1)
        def _():
            out_ref[...] = acc_ref[...]


def _gmm(xs, W1b, W3b, W2b, b3r, te, na):
    def ic(i, na_ref):
        return jnp.minimum(i, na_ref[0] - 1)

    grid_spec = pltpu.PrefetchScalarGridSpec(
        num_scalar_prefetch=2,
        grid=(NT, NJ),
        in_specs=[
            pl.BlockSpec((T, D), lambda i, j, te, na: (ic(i, na), 0)),
            pl.BlockSpec((1, F_BLK, D), lambda i, j, te, na: (te[ic(i, na)], j, 0)),
            pl.BlockSpec((1, F_BLK, D), lambda i, j, te, na: (te[ic(i, na)], j, 0)),
            pl.BlockSpec((1, D, F_BLK), lambda i, j, te, na: (te[ic(i, na)], 0, j)),
            pl.BlockSpec((1, 1, F_BLK), lambda i, j, te, na: (te[ic(i, na)], 0, j)),
        ],
        out_specs=pl.BlockSpec((T, D), lambda i, j, te, na: (ic(i, na), 0)),
        scratch_shapes=[pltpu.VMEM((T, D), jnp.float32),
                        pltpu.VMEM((T, D), jnp.bfloat16)],
    )
    return pl.pallas_call(
        _gmm_body,
        grid_spec=grid_spec,
        out_shape=jax.ShapeDtypeStruct((NP, D), jnp.float32),
        compiler_params=pltpu.CompilerParams(
            dimension_semantics=("arbitrary", "arbitrary")),
    )(te, na, xs, W1b, W3b, W2b, b3r)


_TN = N // NW
_CH2 = 32
_NCH2 = _TN // _CH2


def _combine_sc(rows_mat, pos0, pos1, w0, w1):
    mesh = plsc.VectorSubcoreMesh(core_axis_name="c", subcore_axis_name="s")

    @functools.partial(
        pl.kernel,
        out_type=jax.ShapeDtypeStruct((N, D), jnp.float32),
        mesh=mesh,
        scratch_types=[
            pltpu.VMEM((_CH2,), jnp.int32),
            pltpu.VMEM((_CH2,), jnp.int32),
            pltpu.VMEM((_CH2, 16), jnp.float32),
            pltpu.VMEM((_CH2, 16), jnp.float32),
            pltpu.VMEM((_CH2, D), jnp.float32),
            pltpu.VMEM((_CH2, D), jnp.float32),
            pltpu.SemaphoreType.DMA,
        ],
    )
    def k(rows_hbm, p0_hbm, p1_hbm, w0_hbm, w1_hbm, out_hbm,
          i0v, i1v, w0v, w1v, r0, r1, sem):
        wid = lax.axis_index("s") * 2 + lax.axis_index("c")
        base = wid * _TN

        def chunk(c, carry):
            off = base + c * _CH2
            pltpu.sync_copy(p0_hbm.at[pl.ds(off, _CH2)], i0v)
            pltpu.sync_copy(p1_hbm.at[pl.ds(off, _CH2)], i1v)
            pltpu.sync_copy(w0_hbm.at[pl.ds(off, _CH2)], w0v)
            pltpu.sync_copy(w1_hbm.at[pl.ds(off, _CH2)], w1v)
            pltpu.async_copy(rows_hbm.at[i0v], r0, sem).wait()
            pltpu.async_copy(rows_hbm.at[i1v], r1, sem).wait()

            def tokrow(j, carry2):
                w0s = w0v[j, pl.ds(0, 16)]
                w1s = w1v[j, pl.ds(0, 16)]
                for cc in range(D // 16):
                    sl = pl.ds(cc * 16, 16)
                    r0[j, sl] = r0[j, sl] * w0s + r1[j, sl] * w1s
                return carry2

            lax.fori_loop(0, _CH2, tokrow, 0)
            pltpu.sync_copy(r0, out_hbm.at[pl.ds(off, _CH2)])
            return carry

        lax.fori_loop(0, _NCH2, chunk, 0)

    return k(rows_mat, pos0, pos1, w0, w1)


def kernel(x, W1, W2, W3, b3, Wg):
    xr = x.reshape(N, D)
    b3r = b3.reshape(E, 1, F)

    eidx, w0, w1 = _router(xr, Wg)

    ef = eidx.reshape(1, R)
    onehot = (ef == jnp.arange(E, dtype=jnp.int32)[:, None]).astype(jnp.int32)
    csum = jnp.cumsum(onehot, axis=1)
    counts = csum[:, -1]
    rank = jnp.sum(onehot * (csum - 1), axis=0)
    padded = ((counts + T - 1) // T) * T
    pend = jnp.cumsum(padded)
    poff = pend - padded
    pos = jnp.sum(onehot * poff[:, None], axis=0) + rank
    tile_i = jnp.arange(NT, dtype=jnp.int32)
    te = jnp.minimum(
        jnp.sum((tile_i[None, :] * T >= pend[:, None]).astype(jnp.int32), axis=0),
        E - 1).astype(jnp.int32)
    na = (pend[-1] // T).astype(jnp.int32).reshape(1)
    tok = jnp.arange(R, dtype=jnp.int32) // K
    pos2 = pos.reshape(N, K)
    pos0 = pos2[:, 0]
    pos1 = pos2[:, 1]

    xs = _dispatch_sc(xr, tok, pos)

    rows_mat = _gmm(xs, W1, W3, W2, b3r, te, na)

    out = _combine_sc(rows_mat, pos0, pos1, w0, w1)
    return out.reshape(B, S, D)


# speedup vs baseline: 2.7757x; 1.0725x over previous
import functools

import jax
import jax.numpy as jnp
from jax import lax
from jax.experimental import pallas as pl
from jax.experimental.pallas import tpu as pltpu
from jax.experimental.pallas import tpu_sc as plsc

B, S, D, F, E, K = 4, 2048, 1024, 4096, 8, 2
N = B * S
R = N * K
T = 768
NT = (R + E * (T - 1) + T - 1) // T
NP = NT * T
F_BLK = 1024
NJ = F // F_BLK
NW = 32
RB = 1024


def _router_body(x_ref, wg_ref, ei_ref, w0_ref, w1_ref):
    l = lax.dot_general(x_ref[...].astype(jnp.bfloat16),
                        wg_ref[...].astype(jnp.bfloat16),
                        (((1,), (1,)), ((), ())),
                        preferred_element_type=jnp.float32)
    ie = lax.broadcasted_iota(jnp.int32, (RB, E), 1)
    m1 = jnp.max(l, axis=1, keepdims=True)
    i1 = jnp.min(jnp.where(l == m1, ie, E), axis=1, keepdims=True)
    l2 = jnp.where(ie == i1, -jnp.inf, l)
    m2 = jnp.max(l2, axis=1, keepdims=True)
    i2 = jnp.min(jnp.where(l2 == m2, ie, E), axis=1, keepdims=True)
    t = jnp.exp(m2 - m1)
    s = 1.0 + t
    ei_ref[...] = jnp.concatenate([i1, i2], axis=1)
    w0_ref[...] = jnp.broadcast_to(1.0 / s, (RB, 16))
    w1_ref[...] = jnp.broadcast_to(t / s, (RB, 16))


def _router(xr, Wg):
    return pl.pallas_call(
        _router_body,
        grid=(N // RB,),
        in_specs=[
            pl.BlockSpec((RB, D), lambda i: (i, 0)),
            pl.BlockSpec((E, D), lambda i: (0, 0)),
        ],
        out_specs=[
            pl.BlockSpec((RB, K), lambda i: (i, 0)),
            pl.BlockSpec((RB, 16), lambda i: (i, 0)),
            pl.BlockSpec((RB, 16), lambda i: (i, 0)),
        ],
        out_shape=[
            jax.ShapeDtypeStruct((N, K), jnp.int32),
            jax.ShapeDtypeStruct((N, 16), jnp.float32),
            jax.ShapeDtypeStruct((N, 16), jnp.float32),
        ],
    )(xr, Wg)


_RW = R // NW
_CH = 64
_NCH = _RW // _CH


def _dispatch_sc(xb, tok, pos):
    mesh = plsc.VectorSubcoreMesh(core_axis_name="c", subcore_axis_name="s")

    @functools.partial(
        pl.kernel,
        out_type=jax.ShapeDtypeStruct((NP, D), jnp.float32),
        mesh=mesh,
        scratch_types=[
            pltpu.VMEM((_CH,), jnp.int32),
            pltpu.VMEM((_CH,), jnp.int32),
            pltpu.VMEM((_CH, D), jnp.float32),
            pltpu.SemaphoreType.DMA,
        ],
    )
    def k(xb_hbm, tok_hbm, pos_hbm, xs_hbm, tokv, posv, rows, sem):
        wid = lax.axis_index("s") * 2 + lax.axis_index("c")
        base = wid * _RW

        def body(c, carry):
            off = base + c * _CH
            pltpu.sync_copy(tok_hbm.at[pl.ds(off, _CH)], tokv)
            pltpu.sync_copy(pos_hbm.at[pl.ds(off, _CH)], posv)
            pltpu.async_copy(xb_hbm.at[tokv], rows, sem).wait()
            pltpu.async_copy(rows, xs_hbm.at[posv], sem).wait()
            return carry

        lax.fori_loop(0, _NCH, body, 0)

    return k(xb, tok, pos)


def _gmm_body(te_ref, na_ref, xs_ref, w1_ref, w3_ref, w2_ref, b3_ref,
              out_ref, acc_ref, xb_ref):
    i = pl.program_id(0)
    j = pl.program_id(1)

    @pl.when((i < na_ref[0]) & (j == 0))
    def _():
        xb_ref[...] = xs_ref[...].astype(jnp.bfloat16)

    @pl.when(i < na_ref[0])
    def _():
        xb = xb_ref[...]
        u = lax.dot_general(xb, w1_ref[0].astype(jnp.bfloat16),
                            (((1,), (1,)), ((), ())),
                            preferred_element_type=jnp.float32)
        v = lax.dot_general(xb, w3_ref[0].astype(jnp.bfloat16),
                            (((1,), (1,)), ((), ())),
                            preferred_element_type=jnp.float32)
        h = (u * jax.nn.sigmoid(u)) * (v + b3_ref[0])
        p = lax.dot_general(h.astype(jnp.bfloat16), w2_ref[0].astype(jnp.bfloat16),
                            (((1,), (1,)), ((), ())),
                            preferred_element_type=jnp.float32)

        @pl.when(j == 0)
        def _():
            acc_ref[...] = p

        @pl.when(j > 0)
        def _():
            acc_ref[...] += p

        @pl.when(j == NJ - 1)
        def _():
            out_ref[...] = acc_ref[...]


def _gmm(xs, W1b, W3b, W2b, b3r, te, na):
    def ic(i, na_ref):
        return jnp.minimum(i, na_ref[0] - 1)

    grid_spec = pltpu.PrefetchScalarGridSpec(
        num_scalar_prefetch=2,
        grid=(NT, NJ),
        in_specs=[
            pl.BlockSpec((T, D), lambda i, j, te, na: (ic(i, na), 0)),
            pl.BlockSpec((1, F_BLK, D), lambda i, j, te, na: (te[ic(i, na)], j, 0)),
            pl.BlockSpec((1, F_BLK, D), lambda i, j, te, na: (te[ic(i, na)], j, 0)),
            pl.BlockSpec((1, D, F_BLK), lambda i, j, te, na: (te[ic(i, na)], 0, j)),
            pl.BlockSpec((1, 1, F_BLK), lambda i, j, te, na: (te[ic(i, na)], 0, j)),
        ],
        out_specs=pl.BlockSpec((T, D), lambda i, j, te, na: (ic(i, na), 0)),
        scratch_shapes=[pltpu.VMEM((T, D), jnp.float32),
                        pltpu.VMEM((T, D), jnp.bfloat16)],
    )
    return pl.pallas_call(
        _gmm_body,
        grid_spec=grid_spec,
        out_shape=jax.ShapeDtypeStruct((NP, D), jnp.float32),
        compiler_params=pltpu.CompilerParams(
            dimension_semantics=("arbitrary", "arbitrary")),
    )(te, na, xs, W1b, W3b, W2b, b3r)


_TN = N // NW
_CH2 = 32
_NCH2 = _TN // _CH2


def _combine_sc(rows_mat, pos0, pos1, w0, w1):
    mesh = plsc.VectorSubcoreMesh(core_axis_name="c", subcore_axis_name="s")

    @functools.partial(
        pl.kernel,
        out_type=jax.ShapeDtypeStruct((N, D), jnp.float32),
        mesh=mesh,
        scratch_types=[
            pltpu.VMEM((_CH2,), jnp.int32),
            pltpu.VMEM((_CH2,), jnp.int32),
            pltpu.VMEM((_CH2, 16), jnp.float32),
            pltpu.VMEM((_CH2, 16), jnp.float32),
            pltpu.VMEM((_CH2, D), jnp.float32),
            pltpu.VMEM((_CH2, D), jnp.float32),
            pltpu.SemaphoreType.DMA,
        ],
    )
    def k(rows_hbm, p0_hbm, p1_hbm, w0_hbm, w1_hbm, out_hbm,
          i0v, i1v, w0v, w1v, r0, r1, sem):
        wid = lax.axis_index("s") * 2 + lax.axis_index("c")
        base = wid * _TN

        def chunk(c, carry):
            off = base + c * _CH2
            pltpu.sync_copy(p0_hbm.at[pl.ds(off, _CH2)], i0v)
            pltpu.sync_copy(p1_hbm.at[pl.ds(off, _CH2)], i1v)
            pltpu.sync_copy(w0_hbm.at[pl.ds(off, _CH2)], w0v)
            pltpu.sync_copy(w1_hbm.at[pl.ds(off, _CH2)], w1v)
            pltpu.async_copy(rows_hbm.at[i0v], r0, sem).wait()
            pltpu.async_copy(rows_hbm.at[i1v], r1, sem).wait()

            def tokrow(j, carry2):
                w0s = w0v[j, pl.ds(0, 16)]
                w1s = w1v[j, pl.ds(0, 16)]
                for cc in range(D // 16):
                    sl = pl.ds(cc * 16, 16)
                    r0[j, sl] = r0[j, sl] * w0s + r1[j, sl] * w1s
                return carry2

            lax.fori_loop(0, _CH2, tokrow, 0)
            pltpu.sync_copy(r0, out_hbm.at[pl.ds(off, _CH2)])
            return carry

        lax.fori_loop(0, _NCH2, chunk, 0)

    return k(rows_mat, pos0, pos1, w0, w1)


def kernel(x, W1, W2, W3, b3, Wg):
    xr = x.reshape(N, D)
    b3r = b3.reshape(E, 1, F)

    eidx, w0, w1 = _router(xr, Wg)

    ef = eidx.reshape(1, R)
    onehot = (ef == jnp.arange(E, dtype=jnp.int32)[:, None]).astype(jnp.int32)
    csum = jnp.cumsum(onehot, axis=1)
    counts = csum[:, -1]
    rank = jnp.sum(onehot * (csum - 1), axis=0)
    padded = ((counts + T - 1) // T) * T
    pend = jnp.cumsum(padded)
    poff = pend - padded
    pos = jnp.sum(onehot * poff[:, None], axis=0) + rank
    tile_i = jnp.arange(NT, dtype=jnp.int32)
    te = jnp.minimum(
        jnp.sum((tile_i[None, :] * T >= pend[:, None]).astype(jnp.int32), axis=0),
        E - 1).astype(jnp.int32)
    na = (pend[-1] // T).astype(jnp.int32).reshape(1)
    tok = jnp.arange(R, dtype=jnp.int32) // K
    pos2 = pos.reshape(N, K)
    pos0 = pos2[:, 0]
    pos1 = pos2[:, 1]

    xs = _dispatch_sc(xr, tok, pos)

    rows_mat = _gmm(xs, W1, W3, W2, b3r, te, na)

    out = _combine_sc(rows_mat, pos0, pos1, w0, w1)
    return out.reshape(B, S, D)


# speedup vs baseline: 2.7943x; 1.0067x over previous
import functools

import jax
import jax.numpy as jnp
from jax import lax
from jax.experimental import pallas as pl
from jax.experimental.pallas import tpu as pltpu
from jax.experimental.pallas import tpu_sc as plsc

B, S, D, F, E, K = 4, 2048, 1024, 4096, 8, 2
N = B * S
R = N * K
T = 1024
NT = (R + E * (T - 1) + T - 1) // T
NP = NT * T
F_BLK = 1024
NJ = F // F_BLK
NW = 32
RB = 1024


def _router_body(x_ref, wg_ref, ei_ref, w0_ref, w1_ref):
    l = lax.dot_general(x_ref[...].astype(jnp.bfloat16),
                        wg_ref[...].astype(jnp.bfloat16),
                        (((1,), (1,)), ((), ())),
                        preferred_element_type=jnp.float32)
    ie = lax.broadcasted_iota(jnp.int32, (RB, E), 1)
    m1 = jnp.max(l, axis=1, keepdims=True)
    i1 = jnp.min(jnp.where(l == m1, ie, E), axis=1, keepdims=True)
    l2 = jnp.where(ie == i1, -jnp.inf, l)
    m2 = jnp.max(l2, axis=1, keepdims=True)
    i2 = jnp.min(jnp.where(l2 == m2, ie, E), axis=1, keepdims=True)
    t = jnp.exp(m2 - m1)
    s = 1.0 + t
    ei_ref[...] = jnp.concatenate([i1, i2], axis=1)
    w0_ref[...] = jnp.broadcast_to(1.0 / s, (RB, 16))
    w1_ref[...] = jnp.broadcast_to(t / s, (RB, 16))


def _router(xr, Wg):
    return pl.pallas_call(
        _router_body,
        grid=(N // RB,),
        in_specs=[
            pl.BlockSpec((RB, D), lambda i: (i, 0)),
            pl.BlockSpec((E, D), lambda i: (0, 0)),
        ],
        out_specs=[
            pl.BlockSpec((RB, K), lambda i: (i, 0)),
            pl.BlockSpec((RB, 16), lambda i: (i, 0)),
            pl.BlockSpec((RB, 16), lambda i: (i, 0)),
        ],
        out_shape=[
            jax.ShapeDtypeStruct((N, K), jnp.int32),
            jax.ShapeDtypeStruct((N, 16), jnp.float32),
            jax.ShapeDtypeStruct((N, 16), jnp.float32),
        ],
    )(xr, Wg)


_RW = R // NW
_CH = 64
_NCH = _RW // _CH


def _dispatch_sc(xb, tok, pos):
    mesh = plsc.VectorSubcoreMesh(core_axis_name="c", subcore_axis_name="s")

    @functools.partial(
        pl.kernel,
        out_type=jax.ShapeDtypeStruct((NP, D), jnp.float32),
        mesh=mesh,
        scratch_types=[
            pltpu.VMEM((_CH,), jnp.int32),
            pltpu.VMEM((_CH,), jnp.int32),
            pltpu.VMEM((_CH, D), jnp.float32),
            pltpu.SemaphoreType.DMA,
        ],
    )
    def k(xb_hbm, tok_hbm, pos_hbm, xs_hbm, tokv, posv, rows, sem):
        wid = lax.axis_index("s") * 2 + lax.axis_index("c")
        base = wid * _RW

        def body(c, carry):
            off = base + c * _CH
            pltpu.sync_copy(tok_hbm.at[pl.ds(off, _CH)], tokv)
            pltpu.sync_copy(pos_hbm.at[pl.ds(off, _CH)], posv)
            pltpu.async_copy(xb_hbm.at[tokv], rows, sem).wait()
            pltpu.async_copy(rows, xs_hbm.at[posv], sem).wait()
            return carry

        lax.fori_loop(0, _NCH, body, 0)

    return k(xb, tok, pos)


def _gmm_body(te_ref, na_ref, xs_ref, w1_ref, w3_ref, w2_ref, b3_ref,
              out_ref, acc_ref, xb_ref):
    i = pl.program_id(0)
    j = pl.program_id(1)

    @pl.when((i < na_ref[0]) & (j == 0))
    def _():
        xb_ref[...] = xs_ref[...].astype(jnp.bfloat16)

    @pl.when(i < na_ref[0])
    def _():
        xb = xb_ref[...]
        u = lax.dot_general(xb, w1_ref[0].astype(jnp.bfloat16),
                            (((1,), (1,)), ((), ())),
                            preferred_element_type=jnp.float32)
        v = lax.dot_general(xb, w3_ref[0].astype(jnp.bfloat16),
                            (((1,), (1,)), ((), ())),
                            preferred_element_type=jnp.float32)
        h = (u * jax.nn.sigmoid(u)) * (v + b3_ref[0])
        p = lax.dot_general(h.astype(jnp.bfloat16), w2_ref[0].astype(jnp.bfloat16),
                            (((1,), (1,)), ((), ())),
                            preferred_element_type=jnp.float32)

        @pl.when(j == 0)
        def _():
            acc_ref[...] = p

        @pl.when(j > 0)
        def _():
            acc_ref[...] += p

        @pl.when(j == NJ - 1)
        def _():
            out_ref[...] = acc_ref[...]


def _gmm(xs, W1b, W3b, W2b, b3r, te, na):
    def ic(i, na_ref):
        return jnp.minimum(i, na_ref[0] - 1)

    grid_spec = pltpu.PrefetchScalarGridSpec(
        num_scalar_prefetch=2,
        grid=(NT, NJ),
        in_specs=[
            pl.BlockSpec((T, D), lambda i, j, te, na: (ic(i, na), 0)),
            pl.BlockSpec((1, F_BLK, D), lambda i, j, te, na: (te[ic(i, na)], j, 0)),
            pl.BlockSpec((1, F_BLK, D), lambda i, j, te, na: (te[ic(i, na)], j, 0)),
            pl.BlockSpec((1, D, F_BLK), lambda i, j, te, na: (te[ic(i, na)], 0, j)),
            pl.BlockSpec((1, 1, F_BLK), lambda i, j, te, na: (te[ic(i, na)], 0, j)),
        ],
        out_specs=pl.BlockSpec((T, D), lambda i, j, te, na: (ic(i, na), 0)),
        scratch_shapes=[pltpu.VMEM((T, D), jnp.float32),
                        pltpu.VMEM((T, D), jnp.bfloat16)],
    )
    return pl.pallas_call(
        _gmm_body,
        grid_spec=grid_spec,
        out_shape=jax.ShapeDtypeStruct((NP, D), jnp.float32),
        compiler_params=pltpu.CompilerParams(
            dimension_semantics=("arbitrary", "arbitrary")),
    )(te, na, xs, W1b, W3b, W2b, b3r)


_TN = N // NW
_CH2 = 32
_NCH2 = _TN // _CH2


def _combine_sc(rows_mat, pos0, pos1, w0, w1):
    mesh = plsc.VectorSubcoreMesh(core_axis_name="c", subcore_axis_name="s")

    @functools.partial(
        pl.kernel,
        out_type=jax.ShapeDtypeStruct((N, D), jnp.float32),
        mesh=mesh,
        scratch_types=[
            pltpu.VMEM((_CH2,), jnp.int32),
            pltpu.VMEM((_CH2,), jnp.int32),
            pltpu.VMEM((_CH2, 16), jnp.float32),
            pltpu.VMEM((_CH2, 16), jnp.float32),
            pltpu.VMEM((_CH2, D), jnp.float32),
            pltpu.VMEM((_CH2, D), jnp.float32),
            pltpu.SemaphoreType.DMA,
        ],
    )
    def k(rows_hbm, p0_hbm, p1_hbm, w0_hbm, w1_hbm, out_hbm,
          i0v, i1v, w0v, w1v, r0, r1, sem):
        wid = lax.axis_index("s") * 2 + lax.axis_index("c")
        base = wid * _TN

        def chunk(c, carry):
            off = base + c * _CH2
            pltpu.sync_copy(p0_hbm.at[pl.ds(off, _CH2)], i0v)
            pltpu.sync_copy(p1_hbm.at[pl.ds(off, _CH2)], i1v)
            pltpu.sync_copy(w0_hbm.at[pl.ds(off, _CH2)], w0v)
            pltpu.sync_copy(w1_hbm.at[pl.ds(off, _CH2)], w1v)
            pltpu.async_copy(rows_hbm.at[i0v], r0, sem).wait()
            pltpu.async_copy(rows_hbm.at[i1v], r1, sem).wait()

            def tokrow(j, carry2):
                w0s = w0v[j, pl.ds(0, 16)]
                w1s = w1v[j, pl.ds(0, 16)]
                for cc in range(D // 16):
                    sl = pl.ds(cc * 16, 16)
                    r0[j, sl] = r0[j, sl] * w0s + r1[j, sl] * w1s
                return carry2

            lax.fori_loop(0, _CH2, tokrow, 0)
            pltpu.sync_copy(r0, out_hbm.at[pl.ds(off, _CH2)])
            return carry

        lax.fori_loop(0, _NCH2, chunk, 0)

    return k(rows_mat, pos0, pos1, w0, w1)


def kernel(x, W1, W2, W3, b3, Wg):
    xr = x.reshape(N, D)
    b3r = b3.reshape(E, 1, F)

    eidx, w0, w1 = _router(xr, Wg)

    ef = eidx.reshape(1, R)
    onehot = (ef == jnp.arange(E, dtype=jnp.int32)[:, None]).astype(jnp.int32)
    csum = jnp.cumsum(onehot, axis=1)
    counts = csum[:, -1]
    rank = jnp.sum(onehot * (csum - 1), axis=0)
    padded = ((counts + T - 1) // T) * T
    pend = jnp.cumsum(padded)
    poff = pend - padded
    pos = jnp.sum(onehot * poff[:, None], axis=0) + rank
    tile_i = jnp.arange(NT, dtype=jnp.int32)
    te = jnp.minimum(
        jnp.sum((tile_i[None, :] * T >= pend[:, None]).astype(jnp.int32), axis=0),
        E - 1).astype(jnp.int32)
    na = (pend[-1] // T).astype(jnp.int32).reshape(1)
    tok = jnp.arange(R, dtype=jnp.int32) // K
    pos2 = pos.reshape(N, K)
    pos0 = pos2[:, 0]
    pos1 = pos2[:, 1]

    xs = _dispatch_sc(xr, tok, pos)

    rows_mat = _gmm(xs, W1, W3, W2, b3r, te, na)

    out = _combine_sc(rows_mat, pos0, pos1, w0, w1)
    return out.reshape(B, S, D)


# speedup vs baseline: 2.8072x; 1.0046x over previous
import functools

import jax
import jax.numpy as jnp
from jax import lax
from jax.experimental import pallas as pl
from jax.experimental.pallas import tpu as pltpu
from jax.experimental.pallas import tpu_sc as plsc

B, S, D, F, E, K = 4, 2048, 1024, 4096, 8, 2
N = B * S
R = N * K
T = 1024
NT = (R + E * (T - 1) + T - 1) // T
NP = NT * T
F_BLK = 1024
NJ = F // F_BLK
NW = 32
RB = 1024


def _router_body(x_ref, wg_ref, ei_ref, w0_ref, w1_ref):
    l = lax.dot_general(x_ref[...].astype(jnp.bfloat16),
                        wg_ref[...].astype(jnp.bfloat16),
                        (((1,), (1,)), ((), ())),
                        preferred_element_type=jnp.float32)
    ie = lax.broadcasted_iota(jnp.int32, (RB, E), 1)
    m1 = jnp.max(l, axis=1, keepdims=True)
    i1 = jnp.min(jnp.where(l == m1, ie, E), axis=1, keepdims=True)
    l2 = jnp.where(ie == i1, -jnp.inf, l)
    m2 = jnp.max(l2, axis=1, keepdims=True)
    i2 = jnp.min(jnp.where(l2 == m2, ie, E), axis=1, keepdims=True)
    t = jnp.exp(m2 - m1)
    s = 1.0 + t
    ei_ref[...] = jnp.concatenate([i1, i2], axis=1)
    w0_ref[...] = jnp.broadcast_to(1.0 / s, (RB, 16))
    w1_ref[...] = jnp.broadcast_to(t / s, (RB, 16))


def _router(xr, Wg):
    return pl.pallas_call(
        _router_body,
        grid=(N // RB,),
        in_specs=[
            pl.BlockSpec((RB, D), lambda i: (i, 0)),
            pl.BlockSpec((E, D), lambda i: (0, 0)),
        ],
        out_specs=[
            pl.BlockSpec((RB, K), lambda i: (i, 0)),
            pl.BlockSpec((RB, 16), lambda i: (i, 0)),
            pl.BlockSpec((RB, 16), lambda i: (i, 0)),
        ],
        out_shape=[
            jax.ShapeDtypeStruct((N, K), jnp.int32),
            jax.ShapeDtypeStruct((N, 16), jnp.float32),
            jax.ShapeDtypeStruct((N, 16), jnp.float32),
        ],
    )(xr, Wg)


_RW = R // NW
_CH = 64
_NCH = _RW // _CH


def _dispatch_sc(xr, pos0, pos1):
    mesh = plsc.VectorSubcoreMesh(core_axis_name="c", subcore_axis_name="s")
    tn = N // NW
    tch = _CH // 2
    nch = tn // tch

    @functools.partial(
        pl.kernel,
        out_type=jax.ShapeDtypeStruct((NP, D), jnp.float32),
        mesh=mesh,
        scratch_types=[
            pltpu.VMEM((tch,), jnp.int32),
            pltpu.VMEM((tch,), jnp.int32),
            pltpu.VMEM((tch, D), jnp.float32),
            pltpu.SemaphoreType.DMA,
        ],
    )
    def k(xr_hbm, p0_hbm, p1_hbm, xs_hbm, p0v, p1v, rows, sem):
        wid = lax.axis_index("s") * 2 + lax.axis_index("c")
        base = wid * tn

        def body(c, carry):
            off = base + c * tch
            pltpu.sync_copy(p0_hbm.at[pl.ds(off, tch)], p0v)
            pltpu.sync_copy(p1_hbm.at[pl.ds(off, tch)], p1v)
            pltpu.sync_copy(xr_hbm.at[pl.ds(off, tch)], rows)
            pltpu.async_copy(rows, xs_hbm.at[p0v], sem).wait()
            pltpu.async_copy(rows, xs_hbm.at[p1v], sem).wait()
            return carry

        lax.fori_loop(0, nch, body, 0)

    return k(xr, pos0, pos1)


def _gmm_body(te_ref, na_ref, xs_ref, w1_ref, w3_ref, w2_ref, b3_ref,
              out_ref, acc_ref, xb_ref):
    i = pl.program_id(0)
    j = pl.program_id(1)

    @pl.when((i < na_ref[0]) & (j == 0))
    def _():
        xb_ref[...] = xs_ref[...].astype(jnp.bfloat16)

    @pl.when(i < na_ref[0])
    def _():
        xb = xb_ref[...]
        u = lax.dot_general(xb, w1_ref[0],
                            (((1,), (1,)), ((), ())),
                            preferred_element_type=jnp.float32)
        v = lax.dot_general(xb, w3_ref[0],
                            (((1,), (1,)), ((), ())),
                            preferred_element_type=jnp.float32)
        h = (u * jax.nn.sigmoid(u)) * (v + b3_ref[0])
        p = lax.dot_general(h.astype(jnp.bfloat16), w2_ref[0],
                            (((1,), (1,)), ((), ())),
                            preferred_element_type=jnp.float32)

        @pl.when(j == 0)
        def _():
            acc_ref[...] = p

        @pl.when(j > 0)
        def _():
            acc_ref[...] += p

        @pl.when(j == NJ - 1)
        def _():
            out_ref[...] = acc_ref[...]


def _gmm(xs, W1b, W3b, W2b, b3r, te, na):
    def ic(i, na_ref):
        return jnp.minimum(i, na_ref[0] - 1)

    grid_spec = pltpu.PrefetchScalarGridSpec(
        num_scalar_prefetch=2,
        grid=(NT, NJ),
        in_specs=[
            pl.BlockSpec((T, D), lambda i, j, te, na: (ic(i, na), 0)),
            pl.BlockSpec((1, F_BLK, D), lambda i, j, te, na: (te[ic(i, na)], j, 0)),
            pl.BlockSpec((1, F_BLK, D), lambda i, j, te, na: (te[ic(i, na)], j, 0)),
            pl.BlockSpec((1, D, F_BLK), lambda i, j, te, na: (te[ic(i, na)], 0, j)),
            pl.BlockSpec((1, 1, F_BLK), lambda i, j, te, na: (te[ic(i, na)], 0, j)),
        ],
        out_specs=pl.BlockSpec((T, D), lambda i, j, te, na: (ic(i, na), 0)),
        scratch_shapes=[pltpu.VMEM((T, D), jnp.float32),
                        pltpu.VMEM((T, D), jnp.bfloat16)],
    )
    return pl.pallas_call(
        _gmm_body,
        grid_spec=grid_spec,
        out_shape=jax.ShapeDtypeStruct((NP, D), jnp.float32),
        compiler_params=pltpu.CompilerParams(
            dimension_semantics=("arbitrary", "arbitrary")),
    )(te, na, xs, W1b, W3b, W2b, b3r)


_TN = N // NW
_CH2 = 32
_NCH2 = _TN // _CH2


def _combine_sc(rows_mat, pos0, pos1, w0, w1):
    mesh = plsc.VectorSubcoreMesh(core_axis_name="c", subcore_axis_name="s")

    @functools.partial(
        pl.kernel,
        out_type=jax.ShapeDtypeStruct((N, D), jnp.float32),
        mesh=mesh,
        scratch_types=[
            pltpu.VMEM((_CH2,), jnp.int32),
            pltpu.VMEM((_CH2,), jnp.int32),
            pltpu.VMEM((_CH2, 16), jnp.float32),
            pltpu.VMEM((_CH2, 16), jnp.float32),
            pltpu.VMEM((_CH2, D), jnp.float32),
            pltpu.VMEM((_CH2, D), jnp.float32),
            pltpu.SemaphoreType.DMA,
        ],
    )
    def k(rows_hbm, p0_hbm, p1_hbm, w0_hbm, w1_hbm, out_hbm,
          i0v, i1v, w0v, w1v, r0, r1, sem):
        wid = lax.axis_index("s") * 2 + lax.axis_index("c")
        base = wid * _TN

        def chunk(c, carry):
            off = base + c * _CH2
            pltpu.sync_copy(p0_hbm.at[pl.ds(off, _CH2)], i0v)
            pltpu.sync_copy(p1_hbm.at[pl.ds(off, _CH2)], i1v)
            pltpu.sync_copy(w0_hbm.at[pl.ds(off, _CH2)], w0v)
            pltpu.sync_copy(w1_hbm.at[pl.ds(off, _CH2)], w1v)
            pltpu.async_copy(rows_hbm.at[i0v], r0, sem).wait()
            pltpu.async_copy(rows_hbm.at[i1v], r1, sem).wait()

            def tokrow(j, carry2):
                w0s = w0v[j, pl.ds(0, 16)]
                w1s = w1v[j, pl.ds(0, 16)]
                for cc in range(D // 16):
                    sl = pl.ds(cc * 16, 16)
                    r0[j, sl] = r0[j, sl] * w0s + r1[j, sl] * w1s
                return carry2

            lax.fori_loop(0, _CH2, tokrow, 0)
            pltpu.sync_copy(r0, out_hbm.at[pl.ds(off, _CH2)])
            return carry

        lax.fori_loop(0, _NCH2, chunk, 0)

    return k(rows_mat, pos0, pos1, w0, w1)


def kernel(x, W1, W2, W3, b3, Wg):
    xr = x.reshape(N, D)
    b3r = b3.reshape(E, 1, F)

    eidx, w0, w1 = _router(xr, Wg)

    ef = eidx.reshape(1, R)
    onehot = (ef == jnp.arange(E, dtype=jnp.int32)[:, None]).astype(jnp.int32)
    csum = jnp.cumsum(onehot, axis=1)
    counts = csum[:, -1]
    rank = jnp.sum(onehot * (csum - 1), axis=0)
    padded = ((counts + T - 1) // T) * T
    pend = jnp.cumsum(padded)
    poff = pend - padded
    pos = jnp.sum(onehot * poff[:, None], axis=0) + rank
    tile_i = jnp.arange(NT, dtype=jnp.int32)
    te = jnp.minimum(
        jnp.sum((tile_i[None, :] * T >= pend[:, None]).astype(jnp.int32), axis=0),
        E - 1).astype(jnp.int32)
    na = (pend[-1] // T).astype(jnp.int32).reshape(1)
    pos2 = pos.reshape(N, K)
    pos0 = pos2[:, 0]
    pos1 = pos2[:, 1]

    xs = _dispatch_sc(xr, pos0, pos1)

    rows_mat = _gmm(xs, W1, W3, W2, b3r, te, na)

    out = _combine_sc(rows_mat, pos0, pos1, w0, w1)
    return out.reshape(B, S, D)


# speedup vs baseline: 2.8445x; 1.0133x over previous
import functools

import jax
import jax.numpy as jnp
from jax import lax
from jax.experimental import pallas as pl
from jax.experimental.pallas import tpu as pltpu
from jax.experimental.pallas import tpu_sc as plsc

B, S, D, F, E, K = 4, 2048, 1024, 4096, 8, 2
N = B * S
R = N * K
T = 1024
NT = (R + E * (T - 1) + T - 1) // T
NP = NT * T
F_BLK = 1024
NJ = F // F_BLK
NW = 32
RB = 1024


def _router_body(x_ref, wg_ref, ei_ref, w0_ref, w1_ref):
    l = lax.dot_general(x_ref[...].astype(jnp.bfloat16),
                        wg_ref[...].astype(jnp.bfloat16),
                        (((1,), (1,)), ((), ())),
                        preferred_element_type=jnp.float32)
    ie = lax.broadcasted_iota(jnp.int32, (RB, E), 1)
    m1 = jnp.max(l, axis=1, keepdims=True)
    i1 = jnp.min(jnp.where(l == m1, ie, E), axis=1, keepdims=True)
    l2 = jnp.where(ie == i1, -jnp.inf, l)
    m2 = jnp.max(l2, axis=1, keepdims=True)
    i2 = jnp.min(jnp.where(l2 == m2, ie, E), axis=1, keepdims=True)
    t = jnp.exp(m2 - m1)
    s = 1.0 + t
    ei_ref[...] = jnp.concatenate([i1, i2], axis=1)
    w0_ref[...] = jnp.broadcast_to(1.0 / s, (RB, 16))
    w1_ref[...] = jnp.broadcast_to(t / s, (RB, 16))


def _router(xr, Wg):
    return pl.pallas_call(
        _router_body,
        grid=(N // RB,),
        in_specs=[
            pl.BlockSpec((RB, D), lambda i: (i, 0)),
            pl.BlockSpec((E, D), lambda i: (0, 0)),
        ],
        out_specs=[
            pl.BlockSpec((RB, K), lambda i: (i, 0)),
            pl.BlockSpec((RB, 16), lambda i: (i, 0)),
            pl.BlockSpec((RB, 16), lambda i: (i, 0)),
        ],
        out_shape=[
            jax.ShapeDtypeStruct((N, K), jnp.int32),
            jax.ShapeDtypeStruct((N, 16), jnp.float32),
            jax.ShapeDtypeStruct((N, 16), jnp.float32),
        ],
    )(xr, Wg)


_RW = R // NW
_CH = 64
_NCH = _RW // _CH


def _dispatch_sc(xr, pos0, pos1):
    mesh = plsc.VectorSubcoreMesh(core_axis_name="c", subcore_axis_name="s")
    tn = N // NW
    tch = _CH // 2
    nch = tn // tch

    @functools.partial(
        pl.kernel,
        out_type=jax.ShapeDtypeStruct((NP, D), jnp.float32),
        mesh=mesh,
        scratch_types=[
            pltpu.VMEM((tch,), jnp.int32),
            pltpu.VMEM((tch,), jnp.int32),
            pltpu.VMEM((tch, D), jnp.float32),
            pltpu.SemaphoreType.DMA,
        ],
    )
    def k(xr_hbm, p0_hbm, p1_hbm, xs_hbm, p0v, p1v, rows, sem):
        wid = lax.axis_index("s") * 2 + lax.axis_index("c")
        base = wid * tn

        def body(c, carry):
            off = base + c * tch
            pltpu.sync_copy(p0_hbm.at[pl.ds(off, tch)], p0v)
            pltpu.sync_copy(p1_hbm.at[pl.ds(off, tch)], p1v)
            pltpu.sync_copy(xr_hbm.at[pl.ds(off, tch)], rows)
            pltpu.async_copy(rows, xs_hbm.at[p0v], sem).wait()
            pltpu.async_copy(rows, xs_hbm.at[p1v], sem).wait()
            return carry

        lax.fori_loop(0, nch, body, 0)

    return k(xr, pos0, pos1)


def _gmm_body(te_ref, na_ref, xs_ref, w1_ref, w3_ref, w2_ref, b3_ref,
              out_ref, acc_ref, xb_ref):
    i = pl.program_id(0)
    j = pl.program_id(1)

    @pl.when((i < na_ref[0]) & (j == 0))
    def _():
        xb_ref[...] = xs_ref[...].astype(jnp.bfloat16)

    @pl.when(i < na_ref[0])
    def _():
        xb = xb_ref[...]
        u = lax.dot_general(xb, w1_ref[0],
                            (((1,), (1,)), ((), ())),
                            preferred_element_type=jnp.float32)
        v = lax.dot_general(xb, w3_ref[0],
                            (((1,), (1,)), ((), ())),
                            preferred_element_type=jnp.float32)
        h = (u * jax.nn.sigmoid(u)) * (v + b3_ref[0])
        p = lax.dot_general(h.astype(jnp.bfloat16), w2_ref[0],
                            (((1,), (1,)), ((), ())),
                            preferred_element_type=jnp.float32)

        @pl.when(j == 0)
        def _():
            acc_ref[...] = p

        @pl.when(j > 0)
        def _():
            acc_ref[...] += p

        @pl.when(j == NJ - 1)
        def _():
            out_ref[...] = acc_ref[...]


def _gmm(xs, W1b, W3b, W2b, b3r, te, na):
    def ic(i, na_ref):
        return jnp.minimum(i, na_ref[0] - 1)

    grid_spec = pltpu.PrefetchScalarGridSpec(
        num_scalar_prefetch=2,
        grid=(NT, NJ),
        in_specs=[
            pl.BlockSpec((T, D), lambda i, j, te, na: (ic(i, na), 0)),
            pl.BlockSpec((1, F_BLK, D), lambda i, j, te, na: (te[ic(i, na)], j, 0)),
            pl.BlockSpec((1, F_BLK, D), lambda i, j, te, na: (te[ic(i, na)], j, 0)),
            pl.BlockSpec((1, D, F_BLK), lambda i, j, te, na: (te[ic(i, na)], 0, j)),
            pl.BlockSpec((1, 1, F_BLK), lambda i, j, te, na: (te[ic(i, na)], 0, j)),
        ],
        out_specs=pl.BlockSpec((T, D), lambda i, j, te, na: (ic(i, na), 0)),
        scratch_shapes=[pltpu.VMEM((T, D), jnp.float32),
                        pltpu.VMEM((T, D), jnp.bfloat16)],
    )
    return pl.pallas_call(
        _gmm_body,
        grid_spec=grid_spec,
        out_shape=jax.ShapeDtypeStruct((NP, D), jnp.float32),
        compiler_params=pltpu.CompilerParams(
            dimension_semantics=("arbitrary", "arbitrary")),
    )(te, na, xs, W1b, W3b, W2b, b3r)


_TN = N // NW
_CH2 = 16
_NCH2 = _TN // _CH2


def _combine_sc(rows_mat, pos0, pos1, w0, w1):
    mesh = plsc.VectorSubcoreMesh(core_axis_name="c", subcore_axis_name="s")

    @functools.partial(
        pl.kernel,
        out_type=jax.ShapeDtypeStruct((N, D), jnp.float32),
        mesh=mesh,
        scratch_types=[
            [pltpu.VMEM((_CH2,), jnp.int32),
             pltpu.VMEM((_CH2,), jnp.int32),
             pltpu.VMEM((_CH2, 16), jnp.float32),
             pltpu.VMEM((_CH2, 16), jnp.float32),
             pltpu.VMEM((_CH2, D), jnp.float32),
             pltpu.VMEM((_CH2, D), jnp.float32),
             pltpu.SemaphoreType.DMA],
            [pltpu.VMEM((_CH2,), jnp.int32),
             pltpu.VMEM((_CH2,), jnp.int32),
             pltpu.VMEM((_CH2, 16), jnp.float32),
             pltpu.VMEM((_CH2, 16), jnp.float32),
             pltpu.VMEM((_CH2, D), jnp.float32),
             pltpu.VMEM((_CH2, D), jnp.float32),
             pltpu.SemaphoreType.DMA],
        ],
    )
    def k(rows_hbm, p0_hbm, p1_hbm, w0_hbm, w1_hbm, out_hbm, bufa, bufb):
        wid = lax.axis_index("s") * 2 + lax.axis_index("c")
        base = wid * _TN

        def issue(buf, off):
            i0v, i1v, w0v, w1v, r0, r1, sem = buf
            pltpu.sync_copy(p0_hbm.at[pl.ds(off, _CH2)], i0v)
            pltpu.sync_copy(p1_hbm.at[pl.ds(off, _CH2)], i1v)
            pltpu.sync_copy(w0_hbm.at[pl.ds(off, _CH2)], w0v)
            pltpu.sync_copy(w1_hbm.at[pl.ds(off, _CH2)], w1v)
            h0 = pltpu.async_copy(rows_hbm.at[i0v], r0, sem)
            h1 = pltpu.async_copy(rows_hbm.at[i1v], r1, sem)
            return h0, h1

        def drain_compute(buf, hs, off):
            i0v, i1v, w0v, w1v, r0, r1, sem = buf
            hs[0].wait()
            hs[1].wait()

            def tokrow(j, carry2):
                w0s = w0v[j, pl.ds(0, 16)]
                w1s = w1v[j, pl.ds(0, 16)]
                for cc in range(D // 16):
                    sl = pl.ds(cc * 16, 16)
                    r0[j, sl] = r0[j, sl] * w0s + r1[j, sl] * w1s
                return carry2

            lax.fori_loop(0, _CH2, tokrow, 0)
            pltpu.sync_copy(r0, out_hbm.at[pl.ds(off, _CH2)])

        def pair(c, carry):
            offa = base + (2 * c) * _CH2
            offb = offa + _CH2
            ha = issue(bufa, offa)
            hb = issue(bufb, offb)
            drain_compute(bufa, ha, offa)
            drain_compute(bufb, hb, offb)
            return carry

        lax.fori_loop(0, _NCH2 // 2, pair, 0)

    return k(rows_mat, pos0, pos1, w0, w1)


def kernel(x, W1, W2, W3, b3, Wg):
    xr = x.reshape(N, D)
    b3r = b3.reshape(E, 1, F)

    eidx, w0, w1 = _router(xr, Wg)

    ef = eidx.reshape(1, R)
    onehot = (ef == jnp.arange(E, dtype=jnp.int32)[:, None]).astype(jnp.int32)
    csum = jnp.cumsum(onehot, axis=1)
    counts = csum[:, -1]
    rank = jnp.sum(onehot * (csum - 1), axis=0)
    padded = ((counts + T - 1) // T) * T
    pend = jnp.cumsum(padded)
    poff = pend - padded
    pos = jnp.sum(onehot * poff[:, None], axis=0) + rank
    tile_i = jnp.arange(NT, dtype=jnp.int32)
    te = jnp.minimum(
        jnp.sum((tile_i[None, :] * T >= pend[:, None]).astype(jnp.int32), axis=0),
        E - 1).astype(jnp.int32)
    na = (pend[-1] // T).astype(jnp.int32).reshape(1)
    pos2 = pos.reshape(N, K)
    pos0 = pos2[:, 0]
    pos1 = pos2[:, 1]

    xs = _dispatch_sc(xr, pos0, pos1)

    rows_mat = _gmm(xs, W1, W3, W2, b3r, te, na)

    out = _combine_sc(rows_mat, pos0, pos1, w0, w1)
    return out.reshape(B, S, D)


# speedup vs baseline: 2.8723x; 1.0098x over previous
import functools

import jax
import jax.numpy as jnp
from jax import lax
from jax.experimental import pallas as pl
from jax.experimental.pallas import tpu as pltpu
from jax.experimental.pallas import tpu_sc as plsc

B, S, D, F, E, K = 4, 2048, 1024, 4096, 8, 2
N = B * S
R = N * K
T = 1024
NT = (R + E * (T - 1) + T - 1) // T
NP = NT * T
F_BLK = 1024
NJ = F // F_BLK
NW = 32
RB = 1024


def _router_body(x_ref, wg_ref, ei_ref, w0_ref, w1_ref):
    l = lax.dot_general(x_ref[...].astype(jnp.bfloat16),
                        wg_ref[...].astype(jnp.bfloat16),
                        (((1,), (1,)), ((), ())),
                        preferred_element_type=jnp.float32)
    ie = lax.broadcasted_iota(jnp.int32, (RB, E), 1)
    m1 = jnp.max(l, axis=1, keepdims=True)
    i1 = jnp.min(jnp.where(l == m1, ie, E), axis=1, keepdims=True)
    l2 = jnp.where(ie == i1, -jnp.inf, l)
    m2 = jnp.max(l2, axis=1, keepdims=True)
    i2 = jnp.min(jnp.where(l2 == m2, ie, E), axis=1, keepdims=True)
    t = jnp.exp(m2 - m1)
    s = 1.0 + t
    ei_ref[...] = jnp.concatenate([i1, i2], axis=1)
    w0_ref[...] = jnp.broadcast_to(1.0 / s, (RB, 16))
    w1_ref[...] = jnp.broadcast_to(t / s, (RB, 16))


def _router(xr, Wg):
    return pl.pallas_call(
        _router_body,
        grid=(N // RB,),
        in_specs=[
            pl.BlockSpec((RB, D), lambda i: (i, 0)),
            pl.BlockSpec((E, D), lambda i: (0, 0)),
        ],
        out_specs=[
            pl.BlockSpec((RB, K), lambda i: (i, 0)),
            pl.BlockSpec((RB, 16), lambda i: (i, 0)),
            pl.BlockSpec((RB, 16), lambda i: (i, 0)),
        ],
        out_shape=[
            jax.ShapeDtypeStruct((N, K), jnp.int32),
            jax.ShapeDtypeStruct((N, 16), jnp.float32),
            jax.ShapeDtypeStruct((N, 16), jnp.float32),
        ],
    )(xr, Wg)


_RW = R // NW
_CH = 64
_NCH = _RW // _CH


def _dispatch_sc(xr, pos0, pos1):
    mesh = plsc.VectorSubcoreMesh(core_axis_name="c", subcore_axis_name="s")
    tn = N // NW
    tch = _CH // 2
    nch = tn // tch

    @functools.partial(
        pl.kernel,
        out_type=jax.ShapeDtypeStruct((NP, D), jnp.float32),
        mesh=mesh,
        scratch_types=[
            [pltpu.VMEM((tch,), jnp.int32),
             pltpu.VMEM((tch,), jnp.int32),
             pltpu.VMEM((tch, D), jnp.float32),
             pltpu.SemaphoreType.DMA],
            [pltpu.VMEM((tch,), jnp.int32),
             pltpu.VMEM((tch,), jnp.int32),
             pltpu.VMEM((tch, D), jnp.float32),
             pltpu.SemaphoreType.DMA],
        ],
    )
    def k(xr_hbm, p0_hbm, p1_hbm, xs_hbm, bufa, bufb):
        wid = lax.axis_index("s") * 2 + lax.axis_index("c")
        base = wid * tn

        def issue(buf, off):
            p0v, p1v, rows, sem = buf
            pltpu.sync_copy(p0_hbm.at[pl.ds(off, tch)], p0v)
            pltpu.sync_copy(p1_hbm.at[pl.ds(off, tch)], p1v)
            return pltpu.async_copy(xr_hbm.at[pl.ds(off, tch)], rows, sem)

        def scatter(buf, hr):
            p0v, p1v, rows, sem = buf
            hr.wait()
            h0 = pltpu.async_copy(rows, xs_hbm.at[p0v], sem)
            h1 = pltpu.async_copy(rows, xs_hbm.at[p1v], sem)
            return h0, h1

        def pair(c, carry):
            offa = base + (2 * c) * tch
            offb = offa + tch
            ha = issue(bufa, offa)
            hb = issue(bufb, offb)
            s0a, s1a = scatter(bufa, ha)
            s0b, s1b = scatter(bufb, hb)
            s0a.wait()
            s1a.wait()
            s0b.wait()
            s1b.wait()
            return carry

        lax.fori_loop(0, nch // 2, pair, 0)

    return k(xr, pos0, pos1)


def _gmm_body(te_ref, na_ref, xs_ref, w1_ref, w3_ref, w2_ref, b3_ref,
              out_ref, acc_ref, xb_ref):
    i = pl.program_id(0)
    j = pl.program_id(1)

    @pl.when((i < na_ref[0]) & (j == 0))
    def _():
        xb_ref[...] = xs_ref[...].astype(jnp.bfloat16)

    @pl.when(i < na_ref[0])
    def _():
        xb = xb_ref[...]
        u = lax.dot_general(xb, w1_ref[0],
                            (((1,), (1,)), ((), ())),
                            preferred_element_type=jnp.float32)
        v = lax.dot_general(xb, w3_ref[0],
                            (((1,), (1,)), ((), ())),
                            preferred_element_type=jnp.float32)
        h = (u * jax.nn.sigmoid(u)) * (v + b3_ref[0])
        p = lax.dot_general(h.astype(jnp.bfloat16), w2_ref[0],
                            (((1,), (1,)), ((), ())),
                            preferred_element_type=jnp.float32)

        @pl.when(j == 0)
        def _():
            acc_ref[...] = p

        @pl.when(j > 0)
        def _():
            acc_ref[...] += p

        @pl.when(j == NJ - 1)
        def _():
            out_ref[...] = acc_ref[...]


def _gmm(xs, W1b, W3b, W2b, b3r, te, na):
    def ic(i, na_ref):
        return jnp.minimum(i, na_ref[0] - 1)

    grid_spec = pltpu.PrefetchScalarGridSpec(
        num_scalar_prefetch=2,
        grid=(NT, NJ),
        in_specs=[
            pl.BlockSpec((T, D), lambda i, j, te, na: (ic(i, na), 0)),
            pl.BlockSpec((1, F_BLK, D), lambda i, j, te, na: (te[ic(i, na)], j, 0)),
            pl.BlockSpec((1, F_BLK, D), lambda i, j, te, na: (te[ic(i, na)], j, 0)),
            pl.BlockSpec((1, D, F_BLK), lambda i, j, te, na: (te[ic(i, na)], 0, j)),
            pl.BlockSpec((1, 1, F_BLK), lambda i, j, te, na: (te[ic(i, na)], 0, j)),
        ],
        out_specs=pl.BlockSpec((T, D), lambda i, j, te, na: (ic(i, na), 0)),
        scratch_shapes=[pltpu.VMEM((T, D), jnp.float32),
                        pltpu.VMEM((T, D), jnp.bfloat16)],
    )
    return pl.pallas_call(
        _gmm_body,
        grid_spec=grid_spec,
        out_shape=jax.ShapeDtypeStruct((NP, D), jnp.float32),
        compiler_params=pltpu.CompilerParams(
            dimension_semantics=("arbitrary", "arbitrary")),
    )(te, na, xs, W1b, W3b, W2b, b3r)


_TN = N // NW
_CH2 = 16
_NCH2 = _TN // _CH2


def _combine_sc(rows_mat, pos0, pos1, w0, w1):
    mesh = plsc.VectorSubcoreMesh(core_axis_name="c", subcore_axis_name="s")

    @functools.partial(
        pl.kernel,
        out_type=jax.ShapeDtypeStruct((N, D), jnp.float32),
        mesh=mesh,
        scratch_types=[
            [pltpu.VMEM((_CH2,), jnp.int32),
             pltpu.VMEM((_CH2,), jnp.int32),
             pltpu.VMEM((_CH2, 16), jnp.float32),
             pltpu.VMEM((_CH2, 16), jnp.float32),
             pltpu.VMEM((_CH2, D), jnp.float32),
             pltpu.VMEM((_CH2, D), jnp.float32),
             pltpu.SemaphoreType.DMA],
            [pltpu.VMEM((_CH2,), jnp.int32),
             pltpu.VMEM((_CH2,), jnp.int32),
             pltpu.VMEM((_CH2, 16), jnp.float32),
             pltpu.VMEM((_CH2, 16), jnp.float32),
             pltpu.VMEM((_CH2, D), jnp.float32),
             pltpu.VMEM((_CH2, D), jnp.float32),
             pltpu.SemaphoreType.DMA],
        ],
    )
    def k(rows_hbm, p0_hbm, p1_hbm, w0_hbm, w1_hbm, out_hbm, bufa, bufb):
        wid = lax.axis_index("s") * 2 + lax.axis_index("c")
        base = wid * _TN

        def issue(buf, off):
            i0v, i1v, w0v, w1v, r0, r1, sem = buf
            pltpu.sync_copy(p0_hbm.at[pl.ds(off, _CH2)], i0v)
            pltpu.sync_copy(p1_hbm.at[pl.ds(off, _CH2)], i1v)
            pltpu.sync_copy(w0_hbm.at[pl.ds(off, _CH2)], w0v)
            pltpu.sync_copy(w1_hbm.at[pl.ds(off, _CH2)], w1v)
            h0 = pltpu.async_copy(rows_hbm.at[i0v], r0, sem)
            h1 = pltpu.async_copy(rows_hbm.at[i1v], r1, sem)
            return h0, h1

        def drain_compute(buf, hs, off):
            i0v, i1v, w0v, w1v, r0, r1, sem = buf
            hs[0].wait()
            hs[1].wait()

            def tokrow(j, carry2):
                w0s = w0v[j, pl.ds(0, 16)]
                w1s = w1v[j, pl.ds(0, 16)]
                for cc in range(D // 16):
                    sl = pl.ds(cc * 16, 16)
                    r0[j, sl] = r0[j, sl] * w0s + r1[j, sl] * w1s
                return carry2

            lax.fori_loop(0, _CH2, tokrow, 0)
            pltpu.sync_copy(r0, out_hbm.at[pl.ds(off, _CH2)])

        def pair(c, carry):
            offa = base + (2 * c) * _CH2
            offb = offa + _CH2
            ha = issue(bufa, offa)
            hb = issue(bufb, offb)
            drain_compute(bufa, ha, offa)
            drain_compute(bufb, hb, offb)
            return carry

        lax.fori_loop(0, _NCH2 // 2, pair, 0)

    return k(rows_mat, pos0, pos1, w0, w1)


def kernel(x, W1, W2, W3, b3, Wg):
    xr = x.reshape(N, D)
    b3r = b3.reshape(E, 1, F)

    eidx, w0, w1 = _router(xr, Wg)

    ef = eidx.reshape(1, R)
    onehot = (ef == jnp.arange(E, dtype=jnp.int32)[:, None]).astype(jnp.int32)
    csum = jnp.cumsum(onehot, axis=1)
    counts = csum[:, -1]
    rank = jnp.sum(onehot * (csum - 1), axis=0)
    padded = ((counts + T - 1) // T) * T
    pend = jnp.cumsum(padded)
    poff = pend - padded
    pos = jnp.sum(onehot * poff[:, None], axis=0) + rank
    tile_i = jnp.arange(NT, dtype=jnp.int32)
    te = jnp.minimum(
        jnp.sum((tile_i[None, :] * T >= pend[:, None]).astype(jnp.int32), axis=0),
        E - 1).astype(jnp.int32)
    na = (pend[-1] // T).astype(jnp.int32).reshape(1)
    pos2 = pos.reshape(N, K)
    pos0 = pos2[:, 0]
    pos1 = pos2[:, 1]

    xs = _dispatch_sc(xr, pos0, pos1)

    rows_mat = _gmm(xs, W1, W3, W2, b3r, te, na)

    out = _combine_sc(rows_mat, pos0, pos1, w0, w1)
    return out.reshape(B, S, D)
